# baseline (speedup 1.0000x reference)
import functools

import jax
import jax.numpy as jnp
from jax import lax
from jax.experimental import pallas as pl
from jax.experimental.pallas import tpu as pltpu
from jax.experimental.pallas import tpu_sc as plsc

N = 10000
E = 160000
G = 64
NC, NS, NW = 2, 16, 32
EPAD = 163840
NPAD = 10240
GPAD = 128
IDXB = 128
NCHUNK = 2
ECH = EPAD // NCHUNK


def _fused_mm(A, Wa, *, bias=None, Aextra=(), B=None, Wb=None, D=None,
              D2=None, arow=None, rowmask=None, relu=False, bm=512, dcol=0):
    M, K = A.shape
    No = Wa.shape[1]
    if M < bm:
        bm = M
    assert M % bm == 0
    ops = [A, Wa]
    specs = [pl.BlockSpec((bm, K), lambda i: (i, 0)),
             pl.BlockSpec((K, No), lambda i: (0, 0))]
    tags = ['A', 'W']

    def add(op, spec, tag):
        ops.append(op); specs.append(spec); tags.append(tag)

    for Ax in Aextra:
        add(Ax, pl.BlockSpec((bm, K), lambda i: (i, 0)), 'A2')
    if arow is not None:
        add(arow, pl.BlockSpec((bm, 1), lambda i: (i, 0)), 'arow')
    if B is not None:
        Kb = B.shape[1]
        add(B, pl.BlockSpec((bm, Kb), lambda i: (i, 0)), 'B')
        add(Wb, pl.BlockSpec((Kb, No), lambda i: (0, 0)), 'W')
    if bias is not None:
        add(bias.reshape(1, No), pl.BlockSpec((1, No), lambda i: (0, 0)), 'bias')
    if rowmask is not None:
        add(rowmask, pl.BlockSpec((bm, 1), lambda i: (i, 0)), 'rowmask')
    if D is not None:
        add(D, pl.BlockSpec((bm, No), lambda i, dc=dcol: (i, dc)), 'D')
    if D2 is not None:
        add(D2, pl.BlockSpec((bm, No), lambda i: (i, 0)), 'D')

    def body(*refs):
        rs = list(refs)
        out_ref = rs.pop()
        d = {}
        for t, r in zip(tags, rs):
            d.setdefault(t, []).append(r)
        a = d['A'][0][...]
        for ax in d.get('A2', []):
            a = a + ax[...]
        if 'arow' in d:
            a = a * d['arow'][0][...]
        acc = jnp.dot(a, d['W'][0][...], preferred_element_type=jnp.float32)
        if 'B' in d:
            acc = acc + jnp.dot(d['B'][0][...], d['W'][1][...],
                                preferred_element_type=jnp.float32)
        if 'bias' in d:
            bb = d['bias'][0][...]
            if 'rowmask' in d:
                acc = acc + d['rowmask'][0][...] * bb
            else:
                acc = acc + bb
        for dd in d.get('D', []):
            acc = acc + dd[...]
        if relu:
            acc = jnp.maximum(acc, 0.0)
        out_ref[...] = acc

    return pl.pallas_call(
        body,
        grid=(M // bm,),
        in_specs=specs,
        out_specs=pl.BlockSpec((bm, No), lambda i: (i, 0)),
        out_shape=jax.ShapeDtypeStruct((M, No), jnp.float32),
    )(*ops)


def _edge_mm(Ein, Aw, b1, OUT, W2V, c1, mrows, bm=512, roff=0):
    K = Ein.shape[1]
    M, W = OUT.shape
    wn = W2V.shape[1]
    hp = W - wn
    pre = Aw is None

    def body(*refs):
        if pre:
            ein_ref, out_ref, w2v_ref, c1_ref, eh_ref, h1_ref, g_ref = refs
        else:
            (ein_ref, aw_ref, b1_ref, out_ref, w2v_ref, c1_ref,
             eh_ref, h1_ref, g_ref) = refs
        i = pl.program_id(0)
        o = out_ref[...]
        if pre:
            eh = ein_ref[...] + o[:, wn:]
        else:
            eh = jnp.dot(ein_ref[...], aw_ref[...],
                         preferred_element_type=jnp.float32)
            eh = eh + b1_ref[...] + o[:, wn:]
        eh = jnp.maximum(eh, 0.0)
        eh_ref[...] = eh
        h1 = jnp.dot(eh, w2v_ref[...], preferred_element_type=jnp.float32)
        h1_ref[...] = jnp.maximum(h1 + c1_ref[...] + o[:, :wn], 0.0)
        ridx = lax.broadcasted_iota(jnp.int32, (bm, 1), 0) + i * bm
        ehm = jnp.where(ridx < mrows, eh, 0.0)
        g = lax.dot_general(ehm, ehm, (((0,), (0,)), ((), ())),
                            preferred_element_type=jnp.float32)
        s = jnp.sum(ehm, axis=0, keepdims=True)
        blk = jnp.concatenate([g, s, jnp.zeros((7, hp), jnp.float32)], axis=0)

        @pl.when(i == 0)
        def _():
            g_ref[...] = blk

        @pl.when(i != 0)
        def _():
            g_ref[...] += blk

    nb = roff // bm
    in_specs = [pl.BlockSpec((bm, K), lambda i, o=nb: (i + o, 0))]
    ops = [Ein]
    if not pre:
        in_specs += [pl.BlockSpec((K, hp), lambda i: (0, 0)),
                     pl.BlockSpec((1, hp), lambda i: (0, 0))]
        ops += [Aw, b1.reshape(1, hp)]
    in_specs += [pl.BlockSpec((bm, W), lambda i: (i, 0)),
                 pl.BlockSpec((hp, wn), lambda i: (0, 0)),
                 pl.BlockSpec((1, wn), lambda i: (0, 0))]
    ops += [OUT, W2V, c1.reshape(1, wn)]
    return pl.pallas_call(
        body,
        grid=(M // bm,),
        in_specs=in_specs,
        out_specs=[pl.BlockSpec((bm, hp), lambda i: (i, 0)),
                   pl.BlockSpec((bm, wn), lambda i: (i, 0)),
                   pl.BlockSpec((hp + 8, hp), lambda i: (0, 0))],
        out_shape=[jax.ShapeDtypeStruct((M, hp), jnp.float32),
                   jax.ShapeDtypeStruct((M, wn), jnp.float32),
                   jax.ShapeDtypeStruct((hp + 8, hp), jnp.float32)],
    )(*ops)


def _gram_stats(A, mrows, bm=512):
    M, K = A.shape
    assert M % bm == 0

    def body(a_ref, out_ref):
        i = pl.program_id(0)
        a = a_ref[...]
        ridx = lax.broadcasted_iota(jnp.int32, (bm, 1), 0) + i * bm
        a = jnp.where(ridx < mrows, a, 0.0)
        g = lax.dot_general(a, a, (((0,), (0,)), ((), ())),
                            preferred_element_type=jnp.float32)
        s = jnp.sum(a, axis=0, keepdims=True)
        blk = jnp.concatenate([g, s, jnp.zeros((7, K), jnp.float32)], axis=0)

        @pl.when(i == 0)
        def _():
            out_ref[...] = blk

        @pl.when(i != 0)
        def _():
            out_ref[...] += blk

    return pl.pallas_call(
        body,
        grid=(M // bm,),
        in_specs=[pl.BlockSpec((bm, K), lambda i: (i, 0))],
        out_specs=pl.BlockSpec((K + 8, K), lambda i: (0, 0)),
        out_shape=jax.ShapeDtypeStruct((K + 8, K), jnp.float32),
    )(A)


def _col_stats(A, mrows, bm=1024):
    M, K = A.shape
    assert M % bm == 0

    def body(a_ref, out_ref):
        i = pl.program_id(0)
        a = a_ref[...]
        ridx = lax.broadcasted_iota(jnp.int32, (bm, 1), 0) + i * bm
        a = jnp.where(ridx < mrows, a, 0.0)
        blk = jnp.concatenate([jnp.sum(a, axis=0, keepdims=True),
                               jnp.sum(a * a, axis=0, keepdims=True)], axis=0)

        @pl.when(i == 0)
        def _():
            out_ref[...] = blk

        @pl.when(i != 0)
        def _():
            out_ref[...] += blk

    return pl.pallas_call(
        body,
        grid=(M // bm,),
        in_specs=[pl.BlockSpec((bm, K), lambda i: (i, 0))],
        out_specs=pl.BlockSpec((2, K), lambda i: (0, 0)),
        out_shape=jax.ShapeDtypeStruct((2, K), jnp.float32),
    )(A)


@functools.lru_cache(maxsize=None)
def _sc_gab(V, D, Wn, ech):
    bpw = ech // NW
    bsz = 64
    nb = bpw // bsz
    nk = D // 16
    Wo = Wn + D
    mesh = plsc.VectorSubcoreMesh(core_axis_name="c", subcore_axis_name="s")
    scratch = [pltpu.VMEM((bpw,), jnp.int32), pltpu.VMEM((bpw,), jnp.int32)]
    for _ in range(2):
        scratch += [pltpu.VMEM((bsz, D), jnp.float32),
                    pltpu.VMEM((bsz, Wo), jnp.float32)]
    scratch += [pltpu.SemaphoreType.DMA, pltpu.SemaphoreType.DMA]

    @functools.partial(pl.kernel, mesh=mesh,
                       out_type=jax.ShapeDtypeStruct((ech, Wo), jnp.float32),
                       scratch_types=scratch)
    def k(ta, tbc, rowi_hbm, coli_hbm, outg, rowi, coli,
          a0, bc0, a1, bc1, sem0, sem1):
        sets = ((a0, bc0), (a1, bc1))
        sems = (sem0, sem1)

        c = lax.axis_index("c")
        s = lax.axis_index("s")
        base = (s * NC + c) * bpw
        pltpu.sync_copy(rowi_hbm.at[pl.ds(base, bpw)], rowi)
        pltpu.sync_copy(coli_hbm.at[pl.ds(base, bpw)], coli)

        def descs(j, si):
            a, bc = sets[si]
            rsl = rowi.at[pl.ds(j * bsz, bsz)]
            csl = coli.at[pl.ds(j * bsz, bsz)]
            return [pltpu.make_async_copy(ta.at[rsl], a, sems[si]),
                    pltpu.make_async_copy(tbc.at[csl], bc, sems[si])]

        def start(j, si):
            for d in descs(j, si):
                d.start()

        def finish(j, si):
            a, bc = sets[si]
            for d in descs(j, si):
                d.wait()

            def outer(r, carry):
                r2 = 2 * r
                for rr in (r2, r2 + 1):
                    for kk in range(nk):
                        sa = pl.ds(kk * 16, 16)
                        sb = pl.ds(Wn + kk * 16, 16)
                        bc[rr, sb] = bc[rr, sb] + a[rr, sa]
                return carry

            lax.fori_loop(0, bsz // 2, outer, 0)
            pltpu.sync_copy(bc, outg.at[pl.ds(base + j * bsz, bsz)])

        start(0, 0)

        def body(jj, carry):
            j0 = 2 * jj
            j1 = j0 + 1
            start(j1, 1)
            finish(j0, 0)

            @pl.when(j1 + 1 < nb)
            def _():
                start(j1 + 1, 0)

            finish(j1, 1)
            return carry

        lax.fori_loop(0, nb // 2, body, 0)

    return k


@functools.lru_cache(maxsize=None)
def _sc_scatter_add(bsize, W, npad, mode):
    Wc = W // 2 if mode == 'fsplit' else W
    nworkers = NW if mode in ('esplit2', 'ones') else NS
    per_w = bsize // nworkers
    nb = per_w // IDXB
    rpt = npad // NS
    nout = 2 * npad if mode in ('esplit2', 'ones') else npad
    mesh = plsc.VectorSubcoreMesh(core_axis_name="c", subcore_axis_name="s")

    @functools.partial(
        pl.kernel, mesh=mesh,
        out_type=jax.ShapeDtypeStruct((nout, W), jnp.float32),
        scratch_types=[pltpu.VMEM((nb, IDXB), jnp.int32),
                       pltpu.VMEM((IDXB, Wc), jnp.float32),
                       pltpu.VMEM((IDXB, Wc), jnp.float32),
                       pltpu.VMEM_SHARED((npad, Wc), jnp.float32),
                       pltpu.SemaphoreType.DMA,
                       pltpu.SemaphoreType.DMA],
    )
    def k(vals_hbm, idx3_hbm, zeros_hbm, out_hbm, idx_v, v0, v1, acc,
          sem0, sem1):
        c = lax.axis_index("c")
        s = lax.axis_index("s")
        if mode == 'fsplit':
            coff = c * Wc
            widx = s
            base = s * per_w
            roff = 0
        elif mode in ('esplit2', 'ones'):
            coff = 0
            widx = s * NC + c
            base = widx * per_w
            roff = c * npad
        else:
            coff = 0
            widx = s
            base = s * per_w
            roff = 0

        def phase_zero():
            pltpu.sync_copy(zeros_hbm.at[pl.ds(0, rpt), pl.ds(0, Wc)],
                            acc.at[pl.ds(s * rpt, rpt)])

        def phase_scatter():
            pltpu.sync_copy(idx3_hbm.at[widx], idx_v)

            if mode == 'ones':
                pltpu.sync_copy(vals_hbm, v0)

                def body1(j, carry):
                    pltpu.sync_copy(v0, acc.at[idx_v.at[j]], add=True)
                    return carry

                lax.fori_loop(0, nb, body1, 0)
                return

            def vsrc(j):
                return vals_hbm.at[pl.ds(base + j * IDXB, IDXB),
                                   pl.ds(coff, Wc)]

            def start(j, buf, sem):
                pltpu.async_copy(vsrc(j), buf, sem)

            def finish(j, buf, sem):
                pltpu.make_async_copy(vsrc(j), buf, sem).wait()
                pltpu.sync_copy(buf, acc.at[idx_v.at[j]], add=True)

            start(0, v0, sem0)

            def body(jj, carry):
                j0 = 2 * jj
                j1 = j0 + 1
                start(j1, v1, sem1)
                finish(j0, v0, sem0)

                @pl.when(j1 + 1 < nb)
                def _():
                    start(j1 + 1, v0, sem0)

                finish(j1, v1, sem1)
                return carry

            lax.fori_loop(0, nb // 2, body, 0)
            if nb % 2 == 1:
                finish(nb - 1, v0, sem0)

        def phase_out():
            pltpu.sync_copy(acc.at[pl.ds(s * rpt, rpt)],
                            out_hbm.at[pl.ds(roff + s * rpt, rpt),
                                       pl.ds(coff, Wc)])

        if mode == 'single':
            @pl.when(c == 0)
            def _():
                phase_zero()
            plsc.subcore_barrier()

            @pl.when(c == 0)
            def _():
                phase_scatter()
            plsc.subcore_barrier()

            @pl.when(c == 0)
            def _():
                phase_out()
        else:
            phase_zero()
            plsc.subcore_barrier()
            phase_scatter()
            plsc.subcore_barrier()
            phase_out()

    return k


def _pad_rows(a, mpad, fill=0.0):
    m = a.shape[0]
    if m == mpad:
        return a
    return jnp.concatenate(
        [a, jnp.full((mpad - m,) + a.shape[1:], fill, a.dtype)], axis=0)


def _pad_cols(a, kpad):
    k = a.shape[1]
    if k == kpad:
        return a
    return jnp.concatenate(
        [a, jnp.zeros((a.shape[0], kpad - k), a.dtype)], axis=1)


def _gab(A, BC, row_g, col_g, wn=0):
    return _sc_gab(A.shape[0], A.shape[1], wn, row_g.shape[0])(
        A, BC, row_g, col_g)


def _scatter(vals, idx3, zeros_hbm, npad, mode):
    return _sc_scatter_add(vals.shape[0], vals.shape[1], npad, mode)(
        vals, idx3, zeros_hbm)


def kernel(x, edge_attr, params, edge_index, batch, num_graphs):
    p = params
    row, col = edge_index[0], edge_index[1]

    row_g = _pad_rows(row, EPAD, 0)
    col_g = _pad_rows(col, EPAD, 0)
    row_pad = _pad_rows(row, EPAD, N)
    row_gc = [row_g[k * ECH:(k + 1) * ECH] for k in range(NCHUNK)]
    col_gc = [col_g[k * ECH:(k + 1) * ECH] for k in range(NCHUNK)]
    row_pc = [row_pad[k * ECH:(k + 1) * ECH] for k in range(NCHUNK)]
    row_s3 = [r.reshape(NS, ECH // NS // IDXB, IDXB) for r in row_pc]
    row_w3 = [r.reshape(NW, ECH // NW // IDXB, IDXB) for r in row_pc]
    row_w3f = row_pad.reshape(NW, EPAD // NW // IDXB, IDXB)
    batch_pad = _pad_rows(batch.astype(jnp.int32), NPAD, G)
    bat2_s3 = jnp.concatenate([batch_pad, batch_pad]).reshape(
        NS, 2 * NPAD // NS // IDXB, IDXB)
    nbp = 2 * NCHUNK
    batn_s3 = jnp.concatenate([batch_pad] * nbp).reshape(
        NS, nbp * NPAD // NS // IDXB, IDXB)
    batc_s3 = jnp.concatenate([batch_pad] * NCHUNK).reshape(
        NS, NCHUNK * NPAD // NS // IDXB, IDXB)
    zeros_hbm = jnp.zeros((NPAD, 128), jnp.float32)

    ones_t = jnp.ones((IDXB, 128), jnp.float32)
    ncnt2 = _sc_scatter_add(EPAD, 128, NPAD, 'ones')(
        ones_t, row_w3f, zeros_hbm)
    gcnt128 = _scatter(ncnt2, bat2_s3, zeros_hbm, GPAD, 'single')
    ncnt = ncnt2[:N, 0] + ncnt2[NPAD:NPAD + N, 0]
    gcnt = gcnt128[:G, 0]
    ninv = _pad_rows((1.0 / jnp.maximum(ncnt, 1.0)).reshape(N, 1), NPAD)
    nmask = _pad_rows((ncnt > 0).astype(jnp.float32).reshape(N, 1), NPAD)
    ginv = (1.0 / jnp.maximum(gcnt, 1.0)).reshape(G, 1)
    gmask = (gcnt > 0).astype(jnp.float32).reshape(G, 1)

    x_pad = _pad_rows(x, NPAD)
    u = jnp.zeros((G, 64), jnp.float32)

    def seg_graph(e_chunks):
        if e_chunks[0].shape[1] == 128:
            Se = [_scatter(ec, row_w3[k], zeros_hbm, NPAD, 'esplit2')
                  for k, ec in enumerate(e_chunks)]
            Sg = _scatter(jnp.concatenate(Se), batn_s3, zeros_hbm,
                          GPAD, 'single')
        else:
            Se = [_scatter(ec, row_s3[k], zeros_hbm, NPAD, 'fsplit')
                  for k, ec in enumerate(e_chunks)]
            Sg = _scatter(jnp.concatenate(Se), batc_s3, zeros_hbm,
                          GPAD, 'fsplit')
        return Sg[:G]

    def layer(xp, einc, Aw, Abe, li, sx, tx, last):
        dx = xp.shape[1]
        W1 = p['e%d_1_w' % li]; b1 = p['e%d_1_b' % li]
        h = W1.shape[1]
        hp = max(h, 128)
        Ws, Wd = W1[:dx], W1[dx:2 * dx]
        W2 = p['e%d_2_w' % li]; b2 = p['e%d_2_b' % li]
        Ws_f = _pad_cols(sx[:, None] * Ws, hp)
        Wd_f = _pad_cols(sx[:, None] * Wd, hp)
        b1_f = _pad_cols((b1 + tx @ Ws + tx @ Wd)[None, :], hp)[0] + Abe
        W2_p = _pad_rows(W2, hp)

        As = _fused_mm(xp, Ws_f)
        if last:
            Ad = _fused_mm(xp, Wd_f)
            wn = 0
        else:
            V1 = p['n%d_m1a_w' % li]; c1 = p['n%d_m1a_b' % li]
            Vx, Ve = V1[:dx], V1[dx:]
            wn = Ve.shape[1]
            CAd = _fused_mm(xp, jnp.concatenate([sx[:, None] * Vx, Wd_f],
                                                axis=1))
            W2V = W2_p @ Ve
            c1_f = c1 + tx @ Vx + b2 @ Ve

        e_h_c, S_parts, gram_c = [], [], []
        for kc in range(NCHUNK):
            if last:
                OUT = _gab(As, Ad, row_gc[kc], col_gc[kc])
                e_h = _fused_mm(einc[kc], Aw, bias=b1_f, D=OUT, relu=True)
                e_h_c.append(e_h)
                continue
            OUT = _gab(As, CAd, row_gc[kc], col_gc[kc], wn)
            if Aw is None:
                e_h, h1, gk = _edge_mm(einc[kc], None, None, OUT, W2V,
                                       c1_f, E - kc * ECH, roff=kc * ECH)
            else:
                e_h, h1, gk = _edge_mm(einc[kc], Aw, b1_f, OUT, W2V, c1_f,
                                       E - kc * ECH)
            e_h_c.append(e_h)
            gram_c.append(gk)
            if h1.shape[1] == 128:
                S2 = _scatter(h1, row_w3[kc], zeros_hbm, NPAD, 'esplit2')
                S_parts += [S2[:NPAD], S2[NPAD:]]
            else:
                S_parts.append(_scatter(h1, row_s3[kc], zeros_hbm,
                                        NPAD, 'fsplit'))

        Sg_eh = seg_graph(e_h_c)
        g_agg = _fused_mm(Sg_eh, W2_p, bias=b2, arow=ginv,
                          rowmask=gmask, bm=64)
        if last:
            return None, None, g_agg, (W2_p, b2), None

        V2 = p['n%d_m1b_w' % li]; c2 = p['n%d_m1b_b' % li]
        agg = _fused_mm(S_parts[0], V2, bias=c2, Aextra=tuple(S_parts[1:]),
                        arow=ninv, rowmask=nmask)

        M1 = p['n%d_m2a_w' % li]; d1 = p['n%d_m2a_b' % li]
        Mx, Ma = M1[:dx], M1[dx:]
        Mx_f = sx[:, None] * Mx
        d1_f = d1 + tx @ Mx
        nh = _fused_mm(xp, Mx_f, B=agg, Wb=Ma, bias=d1_f, relu=True)
        n_out = _fused_mm(nh, p['n%d_m2b_w' % li], bias=p['n%d_m2b_b' % li])
        gram = gram_c[0]
        for gk in gram_c[1:]:
            gram = gram + gk
        return n_out, e_h_c, g_agg, (W2_p, b2), gram

    def gmlp(u, g_agg, li):
        G1 = p['g%d_1_w' % li]; gb1 = p['g%d_1_b' % li]
        Gu, Gg = G1[:u.shape[1]], G1[u.shape[1]:]
        uh = _fused_mm(u, Gu, B=g_agg, Wb=Gg, bias=gb1, relu=True, bm=64)
        return _fused_mm(uh, p['g%d_2_w' % li], bias=p['g%d_2_b' % li], bm=64)

    def bn_affine(h_raw, mrows, name):
        st = _col_stats(h_raw, mrows)
        mu = st[0] / mrows
        var = st[1] / mrows - mu * mu
        s = p[name + '_g'] * lax.rsqrt(var + 1e-5)
        t = p[name + '_b'] - mu * s
        return s, t

    def bn_from_gram(gs, W2_p, b2, name):
        K = W2_p.shape[0]
        mu_h = gs[K] / E
        T = gs[:K] @ (W2_p / E)
        muy = mu_h @ W2_p
        var = jnp.sum(W2_p * T, axis=0) - muy * muy
        s = p[name + '_g'] * lax.rsqrt(var + 1e-5)
        t = p[name + '_b'] - (muy + b2) * s
        return s, t

    We1 = p['e1_1_w'][2 * 128:]
    Aw1 = _pad_cols(We1, 128)
    b1f1 = _pad_cols((p['e1_1_b'])[None, :], 128)[0]
    P1 = _fused_mm(edge_attr, Aw1, bias=b1f1, bm=640)
    P1p = _pad_rows(P1, EPAD)
    x1r, eh1_c, g_agg1, (W2p1, b21), gram1 = layer(
        x_pad, [P1p] * NCHUNK, None, jnp.zeros((128,)), 1,
        jnp.ones((128,)), jnp.zeros((128,)), False)
    u = gmlp(u, g_agg1, 1)
    sx1, tx1 = bn_affine(x1r, N, 'bn_n1')
    se1, te1 = bn_from_gram(gram1, W2p1, b21, 'bn_e1')
    We2 = p['e2_1_w'][2 * 256:]
    We2_s = se1[:, None] * We2
    Aw2 = W2p1 @ _pad_cols(We2_s, 128)
    Abe2 = _pad_cols((te1 @ We2 + b21 @ We2_s)[None, :], 128)[0]
    Abe2 = Abe2 + g_agg1[0, 0] * 0.0
    x2r, eh2_c, g_agg2, (W2p2, b22), gram2 = layer(
        x1r, eh1_c, Aw2, Abe2, 2, sx1, tx1, False)
    u = gmlp(u, g_agg2, 2)
    sx2, tx2 = bn_affine(x2r, N, 'bn_n2')
    se2, te2 = bn_from_gram(gram2, W2p2, b22, 'bn_e2')
    We3 = p['e3_1_w'][2 * 512:]
    We3_s = se2[:, None] * We3
    Aw3 = W2p2 @ We3_s
    Abe3 = te2 @ We3 + b22 @ We3_s + g_agg2[0, 0] * 0.0
    _, _, g_agg3, _, _ = layer(x2r, eh2_c, Aw3, Abe3, 3, sx2, tx2, True)
    ea_term = g_agg2 * se2[None, :] + gmask * te2[None, :]
    u = gmlp(u, g_agg3 + ea_term, 3)

    h = _fused_mm(u, p['fc1_w'], bias=p['fc1_b'], relu=True, bm=64)
    return _fused_mm(h, p['fc2_w'], bias=p['fc2_b'], bm=64)


# speedup vs baseline: 1.0423x; 1.0423x over previous
import functools

import jax
import jax.numpy as jnp
from jax import lax
from jax.experimental import pallas as pl
from jax.experimental.pallas import tpu as pltpu
from jax.experimental.pallas import tpu_sc as plsc

N = 10000
E = 160000
G = 64
NC, NS, NW = 2, 16, 32
EPAD = 163840
NPAD = 10240
GPAD = 128
IDXB = 128
NCHUNK = 2
ECH = EPAD // NCHUNK


def _fused_mm(A, Wa, *, bias=None, Aextra=(), B=None, Wb=None, D=None,
              D2=None, arow=None, rowmask=None, relu=False, bm=512, dcol=0):
    M, K = A.shape
    No = Wa.shape[1]
    if M < bm:
        bm = M
    assert M % bm == 0
    ops = [A, Wa]
    specs = [pl.BlockSpec((bm, K), lambda i: (i, 0)),
             pl.BlockSpec((K, No), lambda i: (0, 0))]
    tags = ['A', 'W']

    def add(op, spec, tag):
        ops.append(op); specs.append(spec); tags.append(tag)

    for Ax in Aextra:
        add(Ax, pl.BlockSpec((bm, K), lambda i: (i, 0)), 'A2')
    if arow is not None:
        add(arow, pl.BlockSpec((bm, 1), lambda i: (i, 0)), 'arow')
    if B is not None:
        Kb = B.shape[1]
        add(B, pl.BlockSpec((bm, Kb), lambda i: (i, 0)), 'B')
        add(Wb, pl.BlockSpec((Kb, No), lambda i: (0, 0)), 'W')
    if bias is not None:
        add(bias.reshape(1, No), pl.BlockSpec((1, No), lambda i: (0, 0)), 'bias')
    if rowmask is not None:
        add(rowmask, pl.BlockSpec((bm, 1), lambda i: (i, 0)), 'rowmask')
    if D is not None:
        add(D, pl.BlockSpec((bm, No), lambda i, dc=dcol: (i, dc)), 'D')
    if D2 is not None:
        add(D2, pl.BlockSpec((bm, No), lambda i: (i, 0)), 'D')

    def body(*refs):
        rs = list(refs)
        out_ref = rs.pop()
        d = {}
        for t, r in zip(tags, rs):
            d.setdefault(t, []).append(r)
        a = d['A'][0][...]
        for ax in d.get('A2', []):
            a = a + ax[...]
        if 'arow' in d:
            a = a * d['arow'][0][...]
        acc = jnp.dot(a, d['W'][0][...], preferred_element_type=jnp.float32)
        if 'B' in d:
            acc = acc + jnp.dot(d['B'][0][...], d['W'][1][...],
                                preferred_element_type=jnp.float32)
        if 'bias' in d:
            bb = d['bias'][0][...]
            if 'rowmask' in d:
                acc = acc + d['rowmask'][0][...] * bb
            else:
                acc = acc + bb
        for dd in d.get('D', []):
            acc = acc + dd[...]
        if relu:
            acc = jnp.maximum(acc, 0.0)
        out_ref[...] = acc

    return pl.pallas_call(
        body,
        grid=(M // bm,),
        in_specs=specs,
        out_specs=pl.BlockSpec((bm, No), lambda i: (i, 0)),
        out_shape=jax.ShapeDtypeStruct((M, No), jnp.float32),
    )(*ops)


def _edge_mm(Ein, Aw, b1, OUT, W2V, c1, mrows, bm=512):
    M, K = Ein.shape
    hp = Aw.shape[1]
    wn = W2V.shape[1]

    def body(ein_ref, aw_ref, b1_ref, out_ref, w2v_ref, c1_ref,
             eh_ref, h1_ref, g_ref):
        i = pl.program_id(0)
        o = out_ref[...]
        eh = jnp.dot(ein_ref[...], aw_ref[...],
                     preferred_element_type=jnp.float32)
        eh = jnp.maximum(eh + b1_ref[...] + o[:, wn:], 0.0)
        eh_ref[...] = eh
        h1 = jnp.dot(eh, w2v_ref[...], preferred_element_type=jnp.float32)
        h1_ref[...] = jnp.maximum(h1 + c1_ref[...] + o[:, :wn], 0.0)
        ridx = lax.broadcasted_iota(jnp.int32, (bm, 1), 0) + i * bm
        ehm = jnp.where(ridx < mrows, eh, 0.0)
        g = lax.dot_general(ehm, ehm, (((0,), (0,)), ((), ())),
                            preferred_element_type=jnp.float32)
        s = jnp.sum(ehm, axis=0, keepdims=True)
        blk = jnp.concatenate([g, s, jnp.zeros((7, hp), jnp.float32)], axis=0)

        @pl.when(i == 0)
        def _():
            g_ref[...] = blk

        @pl.when(i != 0)
        def _():
            g_ref[...] += blk

    return pl.pallas_call(
        body,
        grid=(M // bm,),
        in_specs=[pl.BlockSpec((bm, K), lambda i: (i, 0)),
                  pl.BlockSpec((K, hp), lambda i: (0, 0)),
                  pl.BlockSpec((1, hp), lambda i: (0, 0)),
                  pl.BlockSpec((bm, wn + hp), lambda i: (i, 0)),
                  pl.BlockSpec((hp, wn), lambda i: (0, 0)),
                  pl.BlockSpec((1, wn), lambda i: (0, 0))],
        out_specs=[pl.BlockSpec((bm, hp), lambda i: (i, 0)),
                   pl.BlockSpec((bm, wn), lambda i: (i, 0)),
                   pl.BlockSpec((hp + 8, hp), lambda i: (0, 0))],
        out_shape=[jax.ShapeDtypeStruct((M, hp), jnp.float32),
                   jax.ShapeDtypeStruct((M, wn), jnp.float32),
                   jax.ShapeDtypeStruct((hp + 8, hp), jnp.float32)],
    )(Ein, Aw, b1.reshape(1, hp), OUT, W2V, c1.reshape(1, wn))


def _gram_stats(A, mrows, bm=512):
    M, K = A.shape
    assert M % bm == 0

    def body(a_ref, out_ref):
        i = pl.program_id(0)
        a = a_ref[...]
        ridx = lax.broadcasted_iota(jnp.int32, (bm, 1), 0) + i * bm
        a = jnp.where(ridx < mrows, a, 0.0)
        g = lax.dot_general(a, a, (((0,), (0,)), ((), ())),
                            preferred_element_type=jnp.float32)
        s = jnp.sum(a, axis=0, keepdims=True)
        blk = jnp.concatenate([g, s, jnp.zeros((7, K), jnp.float32)], axis=0)

        @pl.when(i == 0)
        def _():
            out_ref[...] = blk

        @pl.when(i != 0)
        def _():
            out_ref[...] += blk

    return pl.pallas_call(
        body,
        grid=(M // bm,),
        in_specs=[pl.BlockSpec((bm, K), lambda i: (i, 0))],
        out_specs=pl.BlockSpec((K + 8, K), lambda i: (0, 0)),
        out_shape=jax.ShapeDtypeStruct((K + 8, K), jnp.float32),
    )(A)


def _col_stats(A, mrows, bm=1024):
    M, K = A.shape
    assert M % bm == 0

    def body(a_ref, out_ref):
        i = pl.program_id(0)
        a = a_ref[...]
        ridx = lax.broadcasted_iota(jnp.int32, (bm, 1), 0) + i * bm
        a = jnp.where(ridx < mrows, a, 0.0)
        blk = jnp.concatenate([jnp.sum(a, axis=0, keepdims=True),
                               jnp.sum(a * a, axis=0, keepdims=True)], axis=0)

        @pl.when(i == 0)
        def _():
            out_ref[...] = blk

        @pl.when(i != 0)
        def _():
            out_ref[...] += blk

    return pl.pallas_call(
        body,
        grid=(M // bm,),
        in_specs=[pl.BlockSpec((bm, K), lambda i: (i, 0))],
        out_specs=pl.BlockSpec((2, K), lambda i: (0, 0)),
        out_shape=jax.ShapeDtypeStruct((2, K), jnp.float32),
    )(A)


@functools.lru_cache(maxsize=None)
def _sc_gab(V, D, Wn, ech):
    bpw = ech // NW
    bsz = 64
    nb = bpw // bsz
    nk = D // 16
    Wo = Wn + D
    mesh = plsc.VectorSubcoreMesh(core_axis_name="c", subcore_axis_name="s")
    scratch = [pltpu.VMEM((bpw,), jnp.int32), pltpu.VMEM((bpw,), jnp.int32)]
    for _ in range(2):
        scratch += [pltpu.VMEM((bsz, D), jnp.float32),
                    pltpu.VMEM((bsz, Wo), jnp.float32)]
    scratch += [pltpu.SemaphoreType.DMA, pltpu.SemaphoreType.DMA]

    @functools.partial(pl.kernel, mesh=mesh,
                       out_type=jax.ShapeDtypeStruct((ech, Wo), jnp.float32),
                       scratch_types=scratch)
    def k(ta, tbc, rowi_hbm, coli_hbm, outg, rowi, coli,
          a0, bc0, a1, bc1, sem0, sem1):
        sets = ((a0, bc0), (a1, bc1))
        sems = (sem0, sem1)

        c = lax.axis_index("c")
        s = lax.axis_index("s")
        base = (s * NC + c) * bpw
        pltpu.sync_copy(rowi_hbm.at[pl.ds(base, bpw)], rowi)
        pltpu.sync_copy(coli_hbm.at[pl.ds(base, bpw)], coli)

        def descs(j, si):
            a, bc = sets[si]
            rsl = rowi.at[pl.ds(j * bsz, bsz)]
            csl = coli.at[pl.ds(j * bsz, bsz)]
            return [pltpu.make_async_copy(ta.at[rsl], a, sems[si]),
                    pltpu.make_async_copy(tbc.at[csl], bc, sems[si])]

        def start(j, si):
            for d in descs(j, si):
                d.start()

        def finish(j, si):
            a, bc = sets[si]
            for d in descs(j, si):
                d.wait()

            def outer(r, carry):
                r2 = 2 * r
                for rr in (r2, r2 + 1):
                    for kk in range(nk):
                        sa = pl.ds(kk * 16, 16)
                        sb = pl.ds(Wn + kk * 16, 16)
                        bc[rr, sb] = bc[rr, sb] + a[rr, sa]
                return carry

            lax.fori_loop(0, bsz // 2, outer, 0)
            pltpu.sync_copy(bc, outg.at[pl.ds(base + j * bsz, bsz)])

        start(0, 0)

        def body(jj, carry):
            j0 = 2 * jj
            j1 = j0 + 1
            start(j1, 1)
            finish(j0, 0)

            @pl.when(j1 + 1 < nb)
            def _():
                start(j1 + 1, 0)

            finish(j1, 1)
            return carry

        lax.fori_loop(0, nb // 2, body, 0)

    return k


@functools.lru_cache(maxsize=None)
def _sc_scatter_add(bsize, W, npad, mode):
    Wc = W // 2 if mode == 'fsplit' else W
    nworkers = NW if mode in ('esplit2', 'ones') else NS
    per_w = bsize // nworkers
    nb = per_w // IDXB
    rpt = npad // NS
    nout = 2 * npad if mode in ('esplit2', 'ones') else npad
    mesh = plsc.VectorSubcoreMesh(core_axis_name="c", subcore_axis_name="s")

    @functools.partial(
        pl.kernel, mesh=mesh,
        out_type=jax.ShapeDtypeStruct((nout, W), jnp.float32),
        scratch_types=[pltpu.VMEM((nb, IDXB), jnp.int32),
                       pltpu.VMEM((IDXB, Wc), jnp.float32),
                       pltpu.VMEM((IDXB, Wc), jnp.float32),
                       pltpu.VMEM_SHARED((npad, Wc), jnp.float32),
                       pltpu.SemaphoreType.DMA,
                       pltpu.SemaphoreType.DMA],
    )
    def k(vals_hbm, idx3_hbm, zeros_hbm, out_hbm, idx_v, v0, v1, acc,
          sem0, sem1):
        c = lax.axis_index("c")
        s = lax.axis_index("s")
        if mode == 'fsplit':
            coff = c * Wc
            widx = s
            base = s * per_w
            roff = 0
        elif mode in ('esplit2', 'ones'):
            coff = 0
            widx = s * NC + c
            base = widx * per_w
            roff = c * npad
        else:
            coff = 0
            widx = s
            base = s * per_w
            roff = 0

        def phase_zero():
            pltpu.sync_copy(zeros_hbm.at[pl.ds(0, rpt), pl.ds(0, Wc)],
                            acc.at[pl.ds(s * rpt, rpt)])

        def phase_scatter():
            pltpu.sync_copy(idx3_hbm.at[widx], idx_v)

            if mode == 'ones':
                pltpu.sync_copy(vals_hbm, v0)

                def body1(j, carry):
                    pltpu.sync_copy(v0, acc.at[idx_v.at[j]], add=True)
                    return carry

                lax.fori_loop(0, nb, body1, 0)
                return

            def vsrc(j):
                return vals_hbm.at[pl.ds(base + j * IDXB, IDXB),
                                   pl.ds(coff, Wc)]

            def start(j, buf, sem):
                pltpu.async_copy(vsrc(j), buf, sem)

            def finish(j, buf, sem):
                pltpu.make_async_copy(vsrc(j), buf, sem).wait()
                pltpu.sync_copy(buf, acc.at[idx_v.at[j]], add=True)

            start(0, v0, sem0)

            def body(jj, carry):
                j0 = 2 * jj
                j1 = j0 + 1
                start(j1, v1, sem1)
                finish(j0, v0, sem0)

                @pl.when(j1 + 1 < nb)
                def _():
                    start(j1 + 1, v0, sem0)

                finish(j1, v1, sem1)
                return carry

            lax.fori_loop(0, nb // 2, body, 0)
            if nb % 2 == 1:
                finish(nb - 1, v0, sem0)

        def phase_out():
            pltpu.sync_copy(acc.at[pl.ds(s * rpt, rpt)],
                            out_hbm.at[pl.ds(roff + s * rpt, rpt),
                                       pl.ds(coff, Wc)])

        if mode == 'single':
            @pl.when(c == 0)
            def _():
                phase_zero()
            plsc.subcore_barrier()

            @pl.when(c == 0)
            def _():
                phase_scatter()
            plsc.subcore_barrier()

            @pl.when(c == 0)
            def _():
                phase_out()
        else:
            phase_zero()
            plsc.subcore_barrier()
            phase_scatter()
            plsc.subcore_barrier()
            phase_out()

    return k


def _pad_rows(a, mpad, fill=0.0):
    m = a.shape[0]
    if m == mpad:
        return a
    return jnp.concatenate(
        [a, jnp.full((mpad - m,) + a.shape[1:], fill, a.dtype)], axis=0)


def _pad_cols(a, kpad):
    k = a.shape[1]
    if k == kpad:
        return a
    return jnp.concatenate(
        [a, jnp.zeros((a.shape[0], kpad - k), a.dtype)], axis=1)


def _gab(A, BC, row_g, col_g, wn=0):
    return _sc_gab(A.shape[0], A.shape[1], wn, row_g.shape[0])(
        A, BC, row_g, col_g)


def _scatter(vals, idx3, zeros_hbm, npad, mode):
    return _sc_scatter_add(vals.shape[0], vals.shape[1], npad, mode)(
        vals, idx3, zeros_hbm)


def kernel(x, edge_attr, params, edge_index, batch, num_graphs):
    p = params
    row, col = edge_index[0], edge_index[1]

    row_g = _pad_rows(row, EPAD, 0)
    col_g = _pad_rows(col, EPAD, 0)
    row_pad = _pad_rows(row, EPAD, N)
    row_gc = [row_g[k * ECH:(k + 1) * ECH] for k in range(NCHUNK)]
    col_gc = [col_g[k * ECH:(k + 1) * ECH] for k in range(NCHUNK)]
    row_pc = [row_pad[k * ECH:(k + 1) * ECH] for k in range(NCHUNK)]
    row_s3 = [r.reshape(NS, ECH // NS // IDXB, IDXB) for r in row_pc]
    row_w3 = [r.reshape(NW, ECH // NW // IDXB, IDXB) for r in row_pc]
    row_w3f = row_pad.reshape(NW, EPAD // NW // IDXB, IDXB)
    batch_pad = _pad_rows(batch.astype(jnp.int32), NPAD, G)
    bat2_s3 = jnp.concatenate([batch_pad, batch_pad]).reshape(
        NS, 2 * NPAD // NS // IDXB, IDXB)
    nbp = 2 * NCHUNK
    batn_s3 = jnp.concatenate([batch_pad] * nbp).reshape(
        NS, nbp * NPAD // NS // IDXB, IDXB)
    batc_s3 = jnp.concatenate([batch_pad] * NCHUNK).reshape(
        NS, NCHUNK * NPAD // NS // IDXB, IDXB)
    zeros_hbm = jnp.zeros((NPAD, 128), jnp.float32)

    ones_t = jnp.ones((IDXB, 128), jnp.float32)
    ncnt2 = _sc_scatter_add(EPAD, 128, NPAD, 'ones')(
        ones_t, row_w3f, zeros_hbm)
    gcnt128 = _scatter(ncnt2, bat2_s3, zeros_hbm, GPAD, 'single')
    ncnt = ncnt2[:N, 0] + ncnt2[NPAD:NPAD + N, 0]
    gcnt = gcnt128[:G, 0]
    ninv = _pad_rows((1.0 / jnp.maximum(ncnt, 1.0)).reshape(N, 1), NPAD)
    nmask = _pad_rows((ncnt > 0).astype(jnp.float32).reshape(N, 1), NPAD)
    ginv = (1.0 / jnp.maximum(gcnt, 1.0)).reshape(G, 1)
    gmask = (gcnt > 0).astype(jnp.float32).reshape(G, 1)

    x_pad = _pad_rows(x, NPAD)
    ea_pad = _pad_rows(edge_attr, EPAD)
    eac = [ea_pad[k * ECH:(k + 1) * ECH] for k in range(NCHUNK)]
    u = jnp.zeros((G, 64), jnp.float32)

    def seg_graph(e_chunks):
        if e_chunks[0].shape[1] == 128:
            Se = [_scatter(ec, row_w3[k], zeros_hbm, NPAD, 'esplit2')
                  for k, ec in enumerate(e_chunks)]
            Sg = _scatter(jnp.concatenate(Se), batn_s3, zeros_hbm,
                          GPAD, 'single')
        else:
            Se = [_scatter(ec, row_s3[k], zeros_hbm, NPAD, 'fsplit')
                  for k, ec in enumerate(e_chunks)]
            Sg = _scatter(jnp.concatenate(Se), batc_s3, zeros_hbm,
                          GPAD, 'fsplit')
        return Sg[:G]

    def layer(xp, einc, Aw, Abe, li, sx, tx, last):
        dx = xp.shape[1]
        W1 = p['e%d_1_w' % li]; b1 = p['e%d_1_b' % li]
        h = W1.shape[1]
        hp = max(h, 128)
        Ws, Wd = W1[:dx], W1[dx:2 * dx]
        W2 = p['e%d_2_w' % li]; b2 = p['e%d_2_b' % li]
        Ws_f = _pad_cols(sx[:, None] * Ws, hp)
        Wd_f = _pad_cols(sx[:, None] * Wd, hp)
        b1_f = _pad_cols((b1 + tx @ Ws + tx @ Wd)[None, :], hp)[0] + Abe
        W2_p = _pad_rows(W2, hp)

        As = _fused_mm(xp, Ws_f)
        if last:
            Ad = _fused_mm(xp, Wd_f)
            wn = 0
        else:
            V1 = p['n%d_m1a_w' % li]; c1 = p['n%d_m1a_b' % li]
            Vx, Ve = V1[:dx], V1[dx:]
            wn = Ve.shape[1]
            CAd = _fused_mm(xp, jnp.concatenate([sx[:, None] * Vx, Wd_f],
                                                axis=1))
            W2V = W2_p @ Ve
            c1_f = c1 + tx @ Vx + b2 @ Ve

        e_h_c, S_parts, gram_c = [], [], []
        for kc in range(NCHUNK):
            if last:
                OUT = _gab(As, Ad, row_gc[kc], col_gc[kc])
                e_h = _fused_mm(einc[kc], Aw, bias=b1_f, D=OUT, relu=True)
                e_h_c.append(e_h)
                continue
            OUT = _gab(As, CAd, row_gc[kc], col_gc[kc], wn)
            e_h, h1, gk = _edge_mm(einc[kc], Aw, b1_f, OUT, W2V, c1_f,
                                   E - kc * ECH)
            e_h_c.append(e_h)
            gram_c.append(gk)
            if h1.shape[1] == 128:
                S2 = _scatter(h1, row_w3[kc], zeros_hbm, NPAD, 'esplit2')
                S_parts += [S2[:NPAD], S2[NPAD:]]
            else:
                S_parts.append(_scatter(h1, row_s3[kc], zeros_hbm,
                                        NPAD, 'fsplit'))

        Sg_eh = seg_graph(e_h_c)
        g_agg = _fused_mm(Sg_eh, W2_p, bias=b2, arow=ginv,
                          rowmask=gmask, bm=64)
        if last:
            return None, None, g_agg, (W2_p, b2), None

        V2 = p['n%d_m1b_w' % li]; c2 = p['n%d_m1b_b' % li]
        agg = _fused_mm(S_parts[0], V2, bias=c2, Aextra=tuple(S_parts[1:]),
                        arow=ninv, rowmask=nmask)

        M1 = p['n%d_m2a_w' % li]; d1 = p['n%d_m2a_b' % li]
        Mx, Ma = M1[:dx], M1[dx:]
        Mx_f = sx[:, None] * Mx
        d1_f = d1 + tx @ Mx
        nh = _fused_mm(xp, Mx_f, B=agg, Wb=Ma, bias=d1_f, relu=True)
        n_out = _fused_mm(nh, p['n%d_m2b_w' % li], bias=p['n%d_m2b_b' % li])
        gram = gram_c[0]
        for gk in gram_c[1:]:
            gram = gram + gk
        return n_out, e_h_c, g_agg, (W2_p, b2), gram

    def gmlp(u, g_agg, li):
        G1 = p['g%d_1_w' % li]; gb1 = p['g%d_1_b' % li]
        Gu, Gg = G1[:u.shape[1]], G1[u.shape[1]:]
        uh = _fused_mm(u, Gu, B=g_agg, Wb=Gg, bias=gb1, relu=True, bm=64)
        return _fused_mm(uh, p['g%d_2_w' % li], bias=p['g%d_2_b' % li], bm=64)

    def bn_affine(h_raw, mrows, name):
        st = _col_stats(h_raw, mrows)
        mu = st[0] / mrows
        var = st[1] / mrows - mu * mu
        s = p[name + '_g'] * lax.rsqrt(var + 1e-5)
        t = p[name + '_b'] - mu * s
        return s, t

    def bn_from_gram(gs, W2_p, b2, name):
        K = W2_p.shape[0]
        mu_h = gs[K] / E
        T = gs[:K] @ (W2_p / E)
        muy = mu_h @ W2_p
        var = jnp.sum(W2_p * T, axis=0) - muy * muy
        s = p[name + '_g'] * lax.rsqrt(var + 1e-5)
        t = p[name + '_b'] - (muy + b2) * s
        return s, t

    We1 = p['e1_1_w'][2 * 128:]
    Aw1 = _pad_cols(We1, 128)
    x1r, eh1_c, g_agg1, (W2p1, b21), gram1 = layer(
        x_pad, eac, Aw1, jnp.zeros((128,)), 1,
        jnp.ones((128,)), jnp.zeros((128,)), False)
    u = gmlp(u, g_agg1, 1)
    sx1, tx1 = bn_affine(x1r, N, 'bn_n1')
    se1, te1 = bn_from_gram(gram1, W2p1, b21, 'bn_e1')
    We2 = p['e2_1_w'][2 * 256:]
    We2_s = se1[:, None] * We2
    Aw2 = W2p1 @ _pad_cols(We2_s, 128)
    Abe2 = _pad_cols((te1 @ We2 + b21 @ We2_s)[None, :], 128)[0]
    Abe2 = Abe2 + g_agg1[0, 0] * 0.0
    x2r, eh2_c, g_agg2, (W2p2, b22), gram2 = layer(
        x1r, eh1_c, Aw2, Abe2, 2, sx1, tx1, False)
    u = gmlp(u, g_agg2, 2)
    sx2, tx2 = bn_affine(x2r, N, 'bn_n2')
    se2, te2 = bn_from_gram(gram2, W2p2, b22, 'bn_e2')
    We3 = p['e3_1_w'][2 * 512:]
    We3_s = se2[:, None] * We3
    Aw3 = W2p2 @ We3_s
    Abe3 = te2 @ We3 + b22 @ We3_s + g_agg2[0, 0] * 0.0
    _, _, g_agg3, _, _ = layer(x2r, eh2_c, Aw3, Abe3, 3, sx2, tx2, True)
    ea_term = g_agg2 * se2[None, :] + gmask * te2[None, :]
    u = gmlp(u, g_agg3 + ea_term, 3)

    h = _fused_mm(u, p['fc1_w'], bias=p['fc1_b'], relu=True, bm=64)
    return _fused_mm(h, p['fc2_w'], bias=p['fc2_b'], bm=64)


# speedup vs baseline: 1.0993x; 1.0546x over previous
import functools

import jax
import jax.numpy as jnp
from jax import lax
from jax.experimental import pallas as pl
from jax.experimental.pallas import tpu as pltpu
from jax.experimental.pallas import tpu_sc as plsc

N = 10000
E = 160000
G = 64
NC, NS, NW = 2, 16, 32
EPAD = 163840
NPAD = 10240
GPAD = 128
IDXB = 128
NCHUNK = 2
ECH = EPAD // NCHUNK


def _fused_mm(A, Wa, *, bias=None, Aextra=(), B=None, Wb=None, D=None,
              D2=None, arow=None, rowmask=None, relu=False, bm=512, dcol=0):
    M, K = A.shape
    No = Wa.shape[1]
    if M < bm:
        bm = M
    assert M % bm == 0
    ops = [A, Wa]
    specs = [pl.BlockSpec((bm, K), lambda i: (i, 0)),
             pl.BlockSpec((K, No), lambda i: (0, 0))]
    tags = ['A', 'W']

    def add(op, spec, tag):
        ops.append(op); specs.append(spec); tags.append(tag)

    for Ax in Aextra:
        add(Ax, pl.BlockSpec((bm, K), lambda i: (i, 0)), 'A2')
    if arow is not None:
        add(arow, pl.BlockSpec((bm, 1), lambda i: (i, 0)), 'arow')
    if B is not None:
        Kb = B.shape[1]
        add(B, pl.BlockSpec((bm, Kb), lambda i: (i, 0)), 'B')
        add(Wb, pl.BlockSpec((Kb, No), lambda i: (0, 0)), 'W')
    if bias is not None:
        add(bias.reshape(1, No), pl.BlockSpec((1, No), lambda i: (0, 0)), 'bias')
    if rowmask is not None:
        add(rowmask, pl.BlockSpec((bm, 1), lambda i: (i, 0)), 'rowmask')
    if D is not None:
        add(D, pl.BlockSpec((bm, No), lambda i, dc=dcol: (i, dc)), 'D')
    if D2 is not None:
        add(D2, pl.BlockSpec((bm, No), lambda i: (i, 0)), 'D')

    def body(*refs):
        rs = list(refs)
        out_ref = rs.pop()
        d = {}
        for t, r in zip(tags, rs):
            d.setdefault(t, []).append(r)
        a = d['A'][0][...]
        for ax in d.get('A2', []):
            a = a + ax[...]
        if 'arow' in d:
            a = a * d['arow'][0][...]
        acc = jnp.dot(a, d['W'][0][...], preferred_element_type=jnp.float32)
        if 'B' in d:
            acc = acc + jnp.dot(d['B'][0][...], d['W'][1][...],
                                preferred_element_type=jnp.float32)
        if 'bias' in d:
            bb = d['bias'][0][...]
            if 'rowmask' in d:
                acc = acc + d['rowmask'][0][...] * bb
            else:
                acc = acc + bb
        for dd in d.get('D', []):
            acc = acc + dd[...]
        if relu:
            acc = jnp.maximum(acc, 0.0)
        out_ref[...] = acc

    return pl.pallas_call(
        body,
        grid=(M // bm,),
        in_specs=specs,
        out_specs=pl.BlockSpec((bm, No), lambda i: (i, 0)),
        out_shape=jax.ShapeDtypeStruct((M, No), jnp.float32),
    )(*ops)


def _edge_mm(Ein, Aw, b1, OUT, W2V, c1, mrows, bm=1024):
    M, K = Ein.shape
    hp = Aw.shape[1]
    wn = W2V.shape[1]

    def body(ein_ref, aw_ref, b1_ref, out_ref, w2v_ref, c1_ref,
             eh_ref, h1_ref, g_ref):
        i = pl.program_id(0)
        o = out_ref[...]
        eh = jnp.dot(ein_ref[...], aw_ref[...],
                     preferred_element_type=jnp.float32)
        eh = jnp.maximum(eh + b1_ref[...] + o[:, wn:], 0.0)
        eh_ref[...] = eh
        h1 = jnp.dot(eh, w2v_ref[...], preferred_element_type=jnp.float32)
        h1_ref[...] = jnp.maximum(h1 + c1_ref[...] + o[:, :wn], 0.0)
        ridx = lax.broadcasted_iota(jnp.int32, (bm, 1), 0) + i * bm
        ehm = jnp.where(ridx < mrows, eh, 0.0)
        g = lax.dot_general(ehm, ehm, (((0,), (0,)), ((), ())),
                            preferred_element_type=jnp.float32)
        s = jnp.sum(ehm, axis=0, keepdims=True)
        blk = jnp.concatenate([g, s, jnp.zeros((7, hp), jnp.float32)], axis=0)

        @pl.when(i == 0)
        def _():
            g_ref[...] = blk

        @pl.when(i != 0)
        def _():
            g_ref[...] += blk

    return pl.pallas_call(
        body,
        grid=(M // bm,),
        in_specs=[pl.BlockSpec((bm, K), lambda i: (i, 0)),
                  pl.BlockSpec((K, hp), lambda i: (0, 0)),
                  pl.BlockSpec((1, hp), lambda i: (0, 0)),
                  pl.BlockSpec((bm, wn + hp), lambda i: (i, 0)),
                  pl.BlockSpec((hp, wn), lambda i: (0, 0)),
                  pl.BlockSpec((1, wn), lambda i: (0, 0))],
        out_specs=[pl.BlockSpec((bm, hp), lambda i: (i, 0)),
                   pl.BlockSpec((bm, wn), lambda i: (i, 0)),
                   pl.BlockSpec((hp + 8, hp), lambda i: (0, 0))],
        out_shape=[jax.ShapeDtypeStruct((M, hp), jnp.float32),
                   jax.ShapeDtypeStruct((M, wn), jnp.float32),
                   jax.ShapeDtypeStruct((hp + 8, hp), jnp.float32)],
    )(Ein, Aw, b1.reshape(1, hp), OUT, W2V, c1.reshape(1, wn))


def _gram_stats(A, mrows, bm=512):
    M, K = A.shape
    assert M % bm == 0

    def body(a_ref, out_ref):
        i = pl.program_id(0)
        a = a_ref[...]
        ridx = lax.broadcasted_iota(jnp.int32, (bm, 1), 0) + i * bm
        a = jnp.where(ridx < mrows, a, 0.0)
        g = lax.dot_general(a, a, (((0,), (0,)), ((), ())),
                            preferred_element_type=jnp.float32)
        s = jnp.sum(a, axis=0, keepdims=True)
        blk = jnp.concatenate([g, s, jnp.zeros((7, K), jnp.float32)], axis=0)

        @pl.when(i == 0)
        def _():
            out_ref[...] = blk

        @pl.when(i != 0)
        def _():
            out_ref[...] += blk

    return pl.pallas_call(
        body,
        grid=(M // bm,),
        in_specs=[pl.BlockSpec((bm, K), lambda i: (i, 0))],
        out_specs=pl.BlockSpec((K + 8, K), lambda i: (0, 0)),
        out_shape=jax.ShapeDtypeStruct((K + 8, K), jnp.float32),
    )(A)


def _col_stats(A, mrows, bm=1024):
    M, K = A.shape
    assert M % bm == 0

    def body(a_ref, out_ref):
        i = pl.program_id(0)
        a = a_ref[...]
        ridx = lax.broadcasted_iota(jnp.int32, (bm, 1), 0) + i * bm
        a = jnp.where(ridx < mrows, a, 0.0)
        blk = jnp.concatenate([jnp.sum(a, axis=0, keepdims=True),
                               jnp.sum(a * a, axis=0, keepdims=True)], axis=0)

        @pl.when(i == 0)
        def _():
            out_ref[...] = blk

        @pl.when(i != 0)
        def _():
            out_ref[...] += blk

    return pl.pallas_call(
        body,
        grid=(M // bm,),
        in_specs=[pl.BlockSpec((bm, K), lambda i: (i, 0))],
        out_specs=pl.BlockSpec((2, K), lambda i: (0, 0)),
        out_shape=jax.ShapeDtypeStruct((2, K), jnp.float32),
    )(A)


@functools.lru_cache(maxsize=None)
def _sc_gab(V, D, Wn, ech):
    bpw = ech // NW
    bsz = 64
    nb = bpw // bsz
    nk = D // 16
    Wo = Wn + D
    mesh = plsc.VectorSubcoreMesh(core_axis_name="c", subcore_axis_name="s")
    scratch = [pltpu.VMEM((bpw,), jnp.int32), pltpu.VMEM((bpw,), jnp.int32)]
    for _ in range(2):
        scratch += [pltpu.VMEM((bsz, D), jnp.float32),
                    pltpu.VMEM((bsz, Wo), jnp.float32)]
    scratch += [pltpu.SemaphoreType.DMA, pltpu.SemaphoreType.DMA]

    @functools.partial(pl.kernel, mesh=mesh,
                       out_type=jax.ShapeDtypeStruct((ech, Wo), jnp.float32),
                       scratch_types=scratch)
    def k(ta, tbc, rowi_hbm, coli_hbm, outg, rowi, coli,
          a0, bc0, a1, bc1, sem0, sem1):
        sets = ((a0, bc0), (a1, bc1))
        sems = (sem0, sem1)

        c = lax.axis_index("c")
        s = lax.axis_index("s")
        base = (s * NC + c) * bpw
        pltpu.sync_copy(rowi_hbm.at[pl.ds(base, bpw)], rowi)
        pltpu.sync_copy(coli_hbm.at[pl.ds(base, bpw)], coli)

        def descs(j, si):
            a, bc = sets[si]
            rsl = rowi.at[pl.ds(j * bsz, bsz)]
            csl = coli.at[pl.ds(j * bsz, bsz)]
            return [pltpu.make_async_copy(ta.at[rsl], a, sems[si]),
                    pltpu.make_async_copy(tbc.at[csl], bc, sems[si])]

        def start(j, si):
            for d in descs(j, si):
                d.start()

        def finish(j, si):
            a, bc = sets[si]
            for d in descs(j, si):
                d.wait()

            def outer(r, carry):
                r2 = 2 * r
                for rr in (r2, r2 + 1):
                    for kk in range(nk):
                        sa = pl.ds(kk * 16, 16)
                        sb = pl.ds(Wn + kk * 16, 16)
                        bc[rr, sb] = bc[rr, sb] + a[rr, sa]
                return carry

            lax.fori_loop(0, bsz // 2, outer, 0)
            pltpu.sync_copy(bc, outg.at[pl.ds(base + j * bsz, bsz)])

        start(0, 0)

        def body(jj, carry):
            j0 = 2 * jj
            j1 = j0 + 1
            start(j1, 1)
            finish(j0, 0)

            @pl.when(j1 + 1 < nb)
            def _():
                start(j1 + 1, 0)

            finish(j1, 1)
            return carry

        lax.fori_loop(0, nb // 2, body, 0)

    return k


@functools.lru_cache(maxsize=None)
def _sc_scatter_add(bsize, W, npad, mode):
    Wc = W // 2 if mode == 'fsplit' else W
    nworkers = NW if mode in ('esplit2', 'ones') else NS
    per_w = bsize // nworkers
    nb = per_w // IDXB
    rpt = npad // NS
    nout = 2 * npad if mode in ('esplit2', 'ones') else npad
    mesh = plsc.VectorSubcoreMesh(core_axis_name="c", subcore_axis_name="s")

    @functools.partial(
        pl.kernel, mesh=mesh,
        out_type=jax.ShapeDtypeStruct((nout, W), jnp.float32),
        scratch_types=[pltpu.VMEM((nb, IDXB), jnp.int32),
                       pltpu.VMEM((IDXB, Wc), jnp.float32),
                       pltpu.VMEM((IDXB, Wc), jnp.float32),
                       pltpu.VMEM_SHARED((npad, Wc), jnp.float32),
                       pltpu.SemaphoreType.DMA,
                       pltpu.SemaphoreType.DMA],
    )
    def k(vals_hbm, idx3_hbm, zeros_hbm, out_hbm, idx_v, v0, v1, acc,
          sem0, sem1):
        c = lax.axis_index("c")
        s = lax.axis_index("s")
        if mode == 'fsplit':
            coff = c * Wc
            widx = s
            base = s * per_w
            roff = 0
        elif mode in ('esplit2', 'ones'):
            coff = 0
            widx = s * NC + c
            base = widx * per_w
            roff = c * npad
        else:
            coff = 0
            widx = s
            base = s * per_w
            roff = 0

        def phase_zero():
            pltpu.sync_copy(zeros_hbm.at[pl.ds(0, rpt), pl.ds(0, Wc)],
                            acc.at[pl.ds(s * rpt, rpt)])

        def phase_scatter():
            pltpu.sync_copy(idx3_hbm.at[widx], idx_v)

            if mode == 'ones':
                pltpu.sync_copy(vals_hbm, v0)

                def body1(j, carry):
                    pltpu.sync_copy(v0, acc.at[idx_v.at[j]], add=True)
                    return carry

                lax.fori_loop(0, nb, body1, 0)
                return

            def vsrc(j):
                return vals_hbm.at[pl.ds(base + j * IDXB, IDXB),
                                   pl.ds(coff, Wc)]

            def start(j, buf, sem):
                pltpu.async_copy(vsrc(j), buf, sem)

            def finish(j, buf, sem):
                pltpu.make_async_copy(vsrc(j), buf, sem).wait()
                pltpu.sync_copy(buf, acc.at[idx_v.at[j]], add=True)

            start(0, v0, sem0)

            def body(jj, carry):
                j0 = 2 * jj
                j1 = j0 + 1
                start(j1, v1, sem1)
                finish(j0, v0, sem0)

                @pl.when(j1 + 1 < nb)
                def _():
                    start(j1 + 1, v0, sem0)

                finish(j1, v1, sem1)
                return carry

            lax.fori_loop(0, nb // 2, body, 0)
            if nb % 2 == 1:
                finish(nb - 1, v0, sem0)

        def phase_out():
            pltpu.sync_copy(acc.at[pl.ds(s * rpt, rpt)],
                            out_hbm.at[pl.ds(roff + s * rpt, rpt),
                                       pl.ds(coff, Wc)])

        if mode == 'single':
            @pl.when(c == 0)
            def _():
                phase_zero()
            plsc.subcore_barrier()

            @pl.when(c == 0)
            def _():
                phase_scatter()
            plsc.subcore_barrier()

            @pl.when(c == 0)
            def _():
                phase_out()
        else:
            phase_zero()
            plsc.subcore_barrier()
            phase_scatter()
            plsc.subcore_barrier()
            phase_out()

    return k


def _pad_rows(a, mpad, fill=0.0):
    m = a.shape[0]
    if m == mpad:
        return a
    return jnp.concatenate(
        [a, jnp.full((mpad - m,) + a.shape[1:], fill, a.dtype)], axis=0)


def _pad_cols(a, kpad):
    k = a.shape[1]
    if k == kpad:
        return a
    return jnp.concatenate(
        [a, jnp.zeros((a.shape[0], kpad - k), a.dtype)], axis=1)


def _gab(A, BC, row_g, col_g, wn=0):
    return _sc_gab(A.shape[0], A.shape[1], wn, row_g.shape[0])(
        A, BC, row_g, col_g)


def _scatter(vals, idx3, zeros_hbm, npad, mode):
    return _sc_scatter_add(vals.shape[0], vals.shape[1], npad, mode)(
        vals, idx3, zeros_hbm)


def kernel(x, edge_attr, params, edge_index, batch, num_graphs):
    p = params
    row, col = edge_index[0], edge_index[1]

    row_g = _pad_rows(row, EPAD, 0)
    col_g = _pad_rows(col, EPAD, 0)
    row_pad = _pad_rows(row, EPAD, N)
    row_gc = [row_g[k * ECH:(k + 1) * ECH] for k in range(NCHUNK)]
    col_gc = [col_g[k * ECH:(k + 1) * ECH] for k in range(NCHUNK)]
    row_pc = [row_pad[k * ECH:(k + 1) * ECH] for k in range(NCHUNK)]
    row_s3 = [r.reshape(NS, ECH // NS // IDXB, IDXB) for r in row_pc]
    row_w3 = [r.reshape(NW, ECH // NW // IDXB, IDXB) for r in row_pc]
    row_w3f = row_pad.reshape(NW, EPAD // NW // IDXB, IDXB)
    batch_pad = _pad_rows(batch.astype(jnp.int32), NPAD, G)
    bat2_s3 = jnp.concatenate([batch_pad, batch_pad]).reshape(
        NS, 2 * NPAD // NS // IDXB, IDXB)
    nbp = 2 * NCHUNK
    batn_s3 = jnp.concatenate([batch_pad] * nbp).reshape(
        NS, nbp * NPAD // NS // IDXB, IDXB)
    batc_s3 = jnp.concatenate([batch_pad] * NCHUNK).reshape(
        NS, NCHUNK * NPAD // NS // IDXB, IDXB)
    zeros_hbm = jnp.zeros((NPAD, 128), jnp.float32)

    ones_t = jnp.ones((IDXB, 128), jnp.float32)
    ncnt2 = _sc_scatter_add(EPAD, 128, NPAD, 'ones')(
        ones_t, row_w3f, zeros_hbm)
    gcnt128 = _scatter(ncnt2, bat2_s3, zeros_hbm, GPAD, 'single')
    ncnt = ncnt2[:N, 0] + ncnt2[NPAD:NPAD + N, 0]
    gcnt = gcnt128[:G, 0]
    ninv = _pad_rows((1.0 / jnp.maximum(ncnt, 1.0)).reshape(N, 1), NPAD)
    nmask = _pad_rows((ncnt > 0).astype(jnp.float32).reshape(N, 1), NPAD)
    ginv = (1.0 / jnp.maximum(gcnt, 1.0)).reshape(G, 1)
    gmask = (gcnt > 0).astype(jnp.float32).reshape(G, 1)

    x_pad = _pad_rows(x, NPAD)
    ea_pad = _pad_rows(edge_attr, EPAD)
    eac = [ea_pad[k * ECH:(k + 1) * ECH] for k in range(NCHUNK)]
    u = jnp.zeros((G, 64), jnp.float32)

    def seg_graph(e_chunks):
        if e_chunks[0].shape[1] == 128:
            Se = [_scatter(ec, row_w3[k], zeros_hbm, NPAD, 'esplit2')
                  for k, ec in enumerate(e_chunks)]
            Sg = _scatter(jnp.concatenate(Se), batn_s3, zeros_hbm,
                          GPAD, 'single')
        else:
            Se = [_scatter(ec, row_s3[k], zeros_hbm, NPAD, 'fsplit')
                  for k, ec in enumerate(e_chunks)]
            Sg = _scatter(jnp.concatenate(Se), batc_s3, zeros_hbm,
                          GPAD, 'fsplit')
        return Sg[:G]

    def layer(xp, einc, Aw, Abe, li, sx, tx, last):
        dx = xp.shape[1]
        W1 = p['e%d_1_w' % li]; b1 = p['e%d_1_b' % li]
        h = W1.shape[1]
        hp = max(h, 128)
        Ws, Wd = W1[:dx], W1[dx:2 * dx]
        W2 = p['e%d_2_w' % li]; b2 = p['e%d_2_b' % li]
        Ws_f = _pad_cols(sx[:, None] * Ws, hp)
        Wd_f = _pad_cols(sx[:, None] * Wd, hp)
        b1_f = _pad_cols((b1 + tx @ Ws + tx @ Wd)[None, :], hp)[0] + Abe
        W2_p = _pad_rows(W2, hp)

        As = _fused_mm(xp, Ws_f)
        if last:
            Ad = _fused_mm(xp, Wd_f)
            wn = 0
        else:
            V1 = p['n%d_m1a_w' % li]; c1 = p['n%d_m1a_b' % li]
            Vx, Ve = V1[:dx], V1[dx:]
            wn = Ve.shape[1]
            CAd = _fused_mm(xp, jnp.concatenate([sx[:, None] * Vx, Wd_f],
                                                axis=1))
            W2V = W2_p @ Ve
            c1_f = c1 + tx @ Vx + b2 @ Ve

        e_h_c, S_parts, gram_c = [], [], []
        for kc in range(NCHUNK):
            if last:
                OUT = _gab(As, Ad, row_gc[kc], col_gc[kc])
                e_h = _fused_mm(einc[kc], Aw, bias=b1_f, D=OUT, relu=True)
                e_h_c.append(e_h)
                continue
            OUT = _gab(As, CAd, row_gc[kc], col_gc[kc], wn)
            e_h, h1, gk = _edge_mm(einc[kc], Aw, b1_f, OUT, W2V, c1_f,
                                   E - kc * ECH)
            e_h_c.append(e_h)
            gram_c.append(gk)
            if h1.shape[1] == 128:
                S2 = _scatter(h1, row_w3[kc], zeros_hbm, NPAD, 'esplit2')
                S_parts += [S2[:NPAD], S2[NPAD:]]
            else:
                S_parts.append(_scatter(h1, row_s3[kc], zeros_hbm,
                                        NPAD, 'fsplit'))

        Sg_eh = seg_graph(e_h_c)
        g_agg = _fused_mm(Sg_eh, W2_p, bias=b2, arow=ginv,
                          rowmask=gmask, bm=64)
        if last:
            return None, None, g_agg, (W2_p, b2), None

        V2 = p['n%d_m1b_w' % li]; c2 = p['n%d_m1b_b' % li]
        agg = _fused_mm(S_parts[0], V2, bias=c2, Aextra=tuple(S_parts[1:]),
                        arow=ninv, rowmask=nmask)

        M1 = p['n%d_m2a_w' % li]; d1 = p['n%d_m2a_b' % li]
        Mx, Ma = M1[:dx], M1[dx:]
        Mx_f = sx[:, None] * Mx
        d1_f = d1 + tx @ Mx
        nh = _fused_mm(xp, Mx_f, B=agg, Wb=Ma, bias=d1_f, relu=True)
        n_out = _fused_mm(nh, p['n%d_m2b_w' % li], bias=p['n%d_m2b_b' % li])
        gram = gram_c[0]
        for gk in gram_c[1:]:
            gram = gram + gk
        return n_out, e_h_c, g_agg, (W2_p, b2), gram

    def gmlp(u, g_agg, li):
        G1 = p['g%d_1_w' % li]; gb1 = p['g%d_1_b' % li]
        Gu, Gg = G1[:u.shape[1]], G1[u.shape[1]:]
        uh = _fused_mm(u, Gu, B=g_agg, Wb=Gg, bias=gb1, relu=True, bm=64)
        return _fused_mm(uh, p['g%d_2_w' % li], bias=p['g%d_2_b' % li], bm=64)

    def bn_affine(h_raw, mrows, name):
        st = _col_stats(h_raw, mrows)
        mu = st[0] / mrows
        var = st[1] / mrows - mu * mu
        s = p[name + '_g'] * lax.rsqrt(var + 1e-5)
        t = p[name + '_b'] - mu * s
        return s, t

    def bn_from_gram(gs, W2_p, b2, name):
        K = W2_p.shape[0]
        mu_h = gs[K] / E
        T = gs[:K] @ (W2_p / E)
        muy = mu_h @ W2_p
        var = jnp.sum(W2_p * T, axis=0) - muy * muy
        s = p[name + '_g'] * lax.rsqrt(var + 1e-5)
        t = p[name + '_b'] - (muy + b2) * s
        return s, t

    We1 = p['e1_1_w'][2 * 128:]
    Aw1 = _pad_cols(We1, 128)
    x1r, eh1_c, g_agg1, (W2p1, b21), gram1 = layer(
        x_pad, eac, Aw1, jnp.zeros((128,)), 1,
        jnp.ones((128,)), jnp.zeros((128,)), False)
    u = gmlp(u, g_agg1, 1)
    sx1, tx1 = bn_affine(x1r, N, 'bn_n1')
    se1, te1 = bn_from_gram(gram1, W2p1, b21, 'bn_e1')
    We2 = p['e2_1_w'][2 * 256:]
    We2_s = se1[:, None] * We2
    Aw2 = W2p1 @ _pad_cols(We2_s, 128)
    Abe2 = _pad_cols((te1 @ We2 + b21 @ We2_s)[None, :], 128)[0]
    Abe2 = Abe2 + g_agg1[0, 0] * 0.0
    x2r, eh2_c, g_agg2, (W2p2, b22), gram2 = layer(
        x1r, eh1_c, Aw2, Abe2, 2, sx1, tx1, False)
    u = gmlp(u, g_agg2, 2)
    sx2, tx2 = bn_affine(x2r, N, 'bn_n2')
    se2, te2 = bn_from_gram(gram2, W2p2, b22, 'bn_e2')
    We3 = p['e3_1_w'][2 * 512:]
    We3_s = se2[:, None] * We3
    Aw3 = W2p2 @ We3_s
    Abe3 = te2 @ We3 + b22 @ We3_s + g_agg2[0, 0] * 0.0
    _, _, g_agg3, _, _ = layer(x2r, eh2_c, Aw3, Abe3, 3, sx2, tx2, True)
    ea_term = g_agg2 * se2[None, :] + gmask * te2[None, :]
    u = gmlp(u, g_agg3 + ea_term, 3)

    h = _fused_mm(u, p['fc1_w'], bias=p['fc1_b'], relu=True, bm=64)
    return _fused_mm(h, p['fc2_w'], bias=p['fc2_b'], bm=64)


# speedup vs baseline: 1.1583x; 1.0537x over previous
import functools

import jax
import jax.numpy as jnp
from jax import lax
from jax.experimental import pallas as pl
from jax.experimental.pallas import tpu as pltpu
from jax.experimental.pallas import tpu_sc as plsc

N = 10000
E = 160000
G = 64
NC, NS, NW = 2, 16, 32
EPAD = 163840
NPAD = 10240
GPAD = 128
IDXB = 128
NCHUNK = 2
ECH = EPAD // NCHUNK


def _fused_mm(A, Wa, *, bias=None, Aextra=(), B=None, Wb=None, D=None,
              D2=None, arow=None, rowmask=None, relu=False, bm=1024, dcol=0):
    M, K = A.shape
    No = Wa.shape[1]
    if M < bm:
        bm = M
    assert M % bm == 0
    ops = [A, Wa]
    specs = [pl.BlockSpec((bm, K), lambda i: (i, 0)),
             pl.BlockSpec((K, No), lambda i: (0, 0))]
    tags = ['A', 'W']

    def add(op, spec, tag):
        ops.append(op); specs.append(spec); tags.append(tag)

    for Ax in Aextra:
        add(Ax, pl.BlockSpec((bm, K), lambda i: (i, 0)), 'A2')
    if arow is not None:
        add(arow, pl.BlockSpec((bm, 1), lambda i: (i, 0)), 'arow')
    if B is not None:
        Kb = B.shape[1]
        add(B, pl.BlockSpec((bm, Kb), lambda i: (i, 0)), 'B')
        add(Wb, pl.BlockSpec((Kb, No), lambda i: (0, 0)), 'W')
    if bias is not None:
        add(bias.reshape(1, No), pl.BlockSpec((1, No), lambda i: (0, 0)), 'bias')
    if rowmask is not None:
        add(rowmask, pl.BlockSpec((bm, 1), lambda i: (i, 0)), 'rowmask')
    if D is not None:
        add(D, pl.BlockSpec((bm, No), lambda i, dc=dcol: (i, dc)), 'D')
    if D2 is not None:
        add(D2, pl.BlockSpec((bm, No), lambda i: (i, 0)), 'D')

    def body(*refs):
        rs = list(refs)
        out_ref = rs.pop()
        d = {}
        for t, r in zip(tags, rs):
            d.setdefault(t, []).append(r)
        a = d['A'][0][...]
        for ax in d.get('A2', []):
            a = a + ax[...]
        if 'arow' in d:
            a = a * d['arow'][0][...]
        acc = jnp.dot(a, d['W'][0][...], preferred_element_type=jnp.float32)
        if 'B' in d:
            acc = acc + jnp.dot(d['B'][0][...], d['W'][1][...],
                                preferred_element_type=jnp.float32)
        if 'bias' in d:
            bb = d['bias'][0][...]
            if 'rowmask' in d:
                acc = acc + d['rowmask'][0][...] * bb
            else:
                acc = acc + bb
        for dd in d.get('D', []):
            acc = acc + dd[...]
        if relu:
            acc = jnp.maximum(acc, 0.0)
        out_ref[...] = acc

    return pl.pallas_call(
        body,
        grid=(M // bm,),
        in_specs=specs,
        out_specs=pl.BlockSpec((bm, No), lambda i: (i, 0)),
        out_shape=jax.ShapeDtypeStruct((M, No), jnp.float32),
    )(*ops)


def _edge_mm(Ein, Aw, b1, OUT, W2V, c1, mrows, bm=2048):
    M, K = Ein.shape
    hp = Aw.shape[1]
    wn = W2V.shape[1]

    def body(ein_ref, aw_ref, b1_ref, out_ref, w2v_ref, c1_ref,
             eh_ref, h1_ref, g_ref):
        i = pl.program_id(0)
        o = out_ref[...]
        eh = jnp.dot(ein_ref[...], aw_ref[...],
                     preferred_element_type=jnp.float32)
        eh = jnp.maximum(eh + b1_ref[...] + o[:, wn:], 0.0)
        eh_ref[...] = eh
        h1 = jnp.dot(eh, w2v_ref[...], preferred_element_type=jnp.float32)
        h1_ref[...] = jnp.maximum(h1 + c1_ref[...] + o[:, :wn], 0.0)
        ridx = lax.broadcasted_iota(jnp.int32, (bm, 1), 0) + i * bm
        ehm = jnp.where(ridx < mrows, eh, 0.0)
        g = lax.dot_general(ehm, ehm, (((0,), (0,)), ((), ())),
                            preferred_element_type=jnp.float32)
        s = jnp.sum(ehm, axis=0, keepdims=True)
        blk = jnp.concatenate([g, s, jnp.zeros((7, hp), jnp.float32)], axis=0)

        @pl.when(i == 0)
        def _():
            g_ref[...] = blk

        @pl.when(i != 0)
        def _():
            g_ref[...] += blk

    return pl.pallas_call(
        body,
        grid=(M // bm,),
        in_specs=[pl.BlockSpec((bm, K), lambda i: (i, 0)),
                  pl.BlockSpec((K, hp), lambda i: (0, 0)),
                  pl.BlockSpec((1, hp), lambda i: (0, 0)),
                  pl.BlockSpec((bm, wn + hp), lambda i: (i, 0)),
                  pl.BlockSpec((hp, wn), lambda i: (0, 0)),
                  pl.BlockSpec((1, wn), lambda i: (0, 0))],
        out_specs=[pl.BlockSpec((bm, hp), lambda i: (i, 0)),
                   pl.BlockSpec((bm, wn), lambda i: (i, 0)),
                   pl.BlockSpec((hp + 8, hp), lambda i: (0, 0))],
        out_shape=[jax.ShapeDtypeStruct((M, hp), jnp.float32),
                   jax.ShapeDtypeStruct((M, wn), jnp.float32),
                   jax.ShapeDtypeStruct((hp + 8, hp), jnp.float32)],
    )(Ein, Aw, b1.reshape(1, hp), OUT, W2V, c1.reshape(1, wn))


def _gram_stats(A, mrows, bm=512):
    M, K = A.shape
    assert M % bm == 0

    def body(a_ref, out_ref):
        i = pl.program_id(0)
        a = a_ref[...]
        ridx = lax.broadcasted_iota(jnp.int32, (bm, 1), 0) + i * bm
        a = jnp.where(ridx < mrows, a, 0.0)
        g = lax.dot_general(a, a, (((0,), (0,)), ((), ())),
                            preferred_element_type=jnp.float32)
        s = jnp.sum(a, axis=0, keepdims=True)
        blk = jnp.concatenate([g, s, jnp.zeros((7, K), jnp.float32)], axis=0)

        @pl.when(i == 0)
        def _():
            out_ref[...] = blk

        @pl.when(i != 0)
        def _():
            out_ref[...] += blk

    return pl.pallas_call(
        body,
        grid=(M // bm,),
        in_specs=[pl.BlockSpec((bm, K), lambda i: (i, 0))],
        out_specs=pl.BlockSpec((K + 8, K), lambda i: (0, 0)),
        out_shape=jax.ShapeDtypeStruct((K + 8, K), jnp.float32),
    )(A)


def _col_stats(A, mrows, bm=1024):
    M, K = A.shape
    assert M % bm == 0

    def body(a_ref, out_ref):
        i = pl.program_id(0)
        a = a_ref[...]
        ridx = lax.broadcasted_iota(jnp.int32, (bm, 1), 0) + i * bm
        a = jnp.where(ridx < mrows, a, 0.0)
        blk = jnp.concatenate([jnp.sum(a, axis=0, keepdims=True),
                               jnp.sum(a * a, axis=0, keepdims=True)], axis=0)

        @pl.when(i == 0)
        def _():
            out_ref[...] = blk

        @pl.when(i != 0)
        def _():
            out_ref[...] += blk

    return pl.pallas_call(
        body,
        grid=(M // bm,),
        in_specs=[pl.BlockSpec((bm, K), lambda i: (i, 0))],
        out_specs=pl.BlockSpec((2, K), lambda i: (0, 0)),
        out_shape=jax.ShapeDtypeStruct((2, K), jnp.float32),
    )(A)


@functools.lru_cache(maxsize=None)
def _sc_gab(V, D, Wn, ech):
    bpw = ech // NW
    bsz = 64
    nb = bpw // bsz
    nk = D // 16
    Wo = Wn + D
    mesh = plsc.VectorSubcoreMesh(core_axis_name="c", subcore_axis_name="s")
    scratch = [pltpu.VMEM((bpw,), jnp.int32), pltpu.VMEM((bpw,), jnp.int32)]
    for _ in range(2):
        scratch += [pltpu.VMEM((bsz, D), jnp.float32),
                    pltpu.VMEM((bsz, Wo), jnp.float32)]
    scratch += [pltpu.SemaphoreType.DMA, pltpu.SemaphoreType.DMA]

    @functools.partial(pl.kernel, mesh=mesh,
                       out_type=jax.ShapeDtypeStruct((ech, Wo), jnp.float32),
                       scratch_types=scratch)
    def k(ta, tbc, rowi_hbm, coli_hbm, outg, rowi, coli,
          a0, bc0, a1, bc1, sem0, sem1):
        sets = ((a0, bc0), (a1, bc1))
        sems = (sem0, sem1)

        c = lax.axis_index("c")
        s = lax.axis_index("s")
        base = (s * NC + c) * bpw
        pltpu.sync_copy(rowi_hbm.at[pl.ds(base, bpw)], rowi)
        pltpu.sync_copy(coli_hbm.at[pl.ds(base, bpw)], coli)

        def descs(j, si):
            a, bc = sets[si]
            rsl = rowi.at[pl.ds(j * bsz, bsz)]
            csl = coli.at[pl.ds(j * bsz, bsz)]
            return [pltpu.make_async_copy(ta.at[rsl], a, sems[si]),
                    pltpu.make_async_copy(tbc.at[csl], bc, sems[si])]

        def start(j, si):
            for d in descs(j, si):
                d.start()

        def finish(j, si):
            a, bc = sets[si]
            for d in descs(j, si):
                d.wait()

            def outer(r, carry):
                r2 = 2 * r
                for rr in (r2, r2 + 1):
                    for kk in range(nk):
                        sa = pl.ds(kk * 16, 16)
                        sb = pl.ds(Wn + kk * 16, 16)
                        bc[rr, sb] = bc[rr, sb] + a[rr, sa]
                return carry

            lax.fori_loop(0, bsz // 2, outer, 0)
            pltpu.sync_copy(bc, outg.at[pl.ds(base + j * bsz, bsz)])

        start(0, 0)

        def body(jj, carry):
            j0 = 2 * jj
            j1 = j0 + 1
            start(j1, 1)
            finish(j0, 0)

            @pl.when(j1 + 1 < nb)
            def _():
                start(j1 + 1, 0)

            finish(j1, 1)
            return carry

        lax.fori_loop(0, nb // 2, body, 0)

    return k


@functools.lru_cache(maxsize=None)
def _sc_scatter_add(bsize, W, npad, mode):
    Wc = W // 2 if mode == 'fsplit' else W
    nworkers = NW if mode in ('esplit2', 'ones') else NS
    per_w = bsize // nworkers
    nb = per_w // IDXB
    rpt = npad // NS
    nout = 2 * npad if mode in ('esplit2', 'ones') else npad
    mesh = plsc.VectorSubcoreMesh(core_axis_name="c", subcore_axis_name="s")

    @functools.partial(
        pl.kernel, mesh=mesh,
        out_type=jax.ShapeDtypeStruct((nout, W), jnp.float32),
        scratch_types=[pltpu.VMEM((nb, IDXB), jnp.int32),
                       pltpu.VMEM((IDXB, Wc), jnp.float32),
                       pltpu.VMEM((IDXB, Wc), jnp.float32),
                       pltpu.VMEM_SHARED((npad, Wc), jnp.float32),
                       pltpu.SemaphoreType.DMA,
                       pltpu.SemaphoreType.DMA],
    )
    def k(vals_hbm, idx3_hbm, zeros_hbm, out_hbm, idx_v, v0, v1, acc,
          sem0, sem1):
        c = lax.axis_index("c")
        s = lax.axis_index("s")
        if mode == 'fsplit':
            coff = c * Wc
            widx = s
            base = s * per_w
            roff = 0
        elif mode in ('esplit2', 'ones'):
            coff = 0
            widx = s * NC + c
            base = widx * per_w
            roff = c * npad
        else:
            coff = 0
            widx = s
            base = s * per_w
            roff = 0

        def phase_zero():
            pltpu.sync_copy(zeros_hbm.at[pl.ds(0, rpt), pl.ds(0, Wc)],
                            acc.at[pl.ds(s * rpt, rpt)])

        def phase_scatter():
            pltpu.sync_copy(idx3_hbm.at[widx], idx_v)

            if mode == 'ones':
                pltpu.sync_copy(vals_hbm, v0)

                def body1(j, carry):
                    pltpu.sync_copy(v0, acc.at[idx_v.at[j]], add=True)
                    return carry

                lax.fori_loop(0, nb, body1, 0)
                return

            def vsrc(j):
                return vals_hbm.at[pl.ds(base + j * IDXB, IDXB),
                                   pl.ds(coff, Wc)]

            def start(j, buf, sem):
                pltpu.async_copy(vsrc(j), buf, sem)

            def finish(j, buf, sem):
                pltpu.make_async_copy(vsrc(j), buf, sem).wait()
                pltpu.sync_copy(buf, acc.at[idx_v.at[j]], add=True)

            start(0, v0, sem0)

            def body(jj, carry):
                j0 = 2 * jj
                j1 = j0 + 1
                start(j1, v1, sem1)
                finish(j0, v0, sem0)

                @pl.when(j1 + 1 < nb)
                def _():
                    start(j1 + 1, v0, sem0)

                finish(j1, v1, sem1)
                return carry

            lax.fori_loop(0, nb // 2, body, 0)
            if nb % 2 == 1:
                finish(nb - 1, v0, sem0)

        def phase_out():
            pltpu.sync_copy(acc.at[pl.ds(s * rpt, rpt)],
                            out_hbm.at[pl.ds(roff + s * rpt, rpt),
                                       pl.ds(coff, Wc)])

        if mode == 'single':
            @pl.when(c == 0)
            def _():
                phase_zero()
            plsc.subcore_barrier()

            @pl.when(c == 0)
            def _():
                phase_scatter()
            plsc.subcore_barrier()

            @pl.when(c == 0)
            def _():
                phase_out()
        else:
            phase_zero()
            plsc.subcore_barrier()
            phase_scatter()
            plsc.subcore_barrier()
            phase_out()

    return k


def _pad_rows(a, mpad, fill=0.0):
    m = a.shape[0]
    if m == mpad:
        return a
    return jnp.concatenate(
        [a, jnp.full((mpad - m,) + a.shape[1:], fill, a.dtype)], axis=0)


def _pad_cols(a, kpad):
    k = a.shape[1]
    if k == kpad:
        return a
    return jnp.concatenate(
        [a, jnp.zeros((a.shape[0], kpad - k), a.dtype)], axis=1)


def _gab(A, BC, row_g, col_g, wn=0):
    return _sc_gab(A.shape[0], A.shape[1], wn, row_g.shape[0])(
        A, BC, row_g, col_g)


def _scatter(vals, idx3, zeros_hbm, npad, mode):
    return _sc_scatter_add(vals.shape[0], vals.shape[1], npad, mode)(
        vals, idx3, zeros_hbm)


def kernel(x, edge_attr, params, edge_index, batch, num_graphs):
    p = params
    row, col = edge_index[0], edge_index[1]

    row_g = _pad_rows(row, EPAD, 0)
    col_g = _pad_rows(col, EPAD, 0)
    row_pad = _pad_rows(row, EPAD, N)
    row_gc = [row_g[k * ECH:(k + 1) * ECH] for k in range(NCHUNK)]
    col_gc = [col_g[k * ECH:(k + 1) * ECH] for k in range(NCHUNK)]
    row_pc = [row_pad[k * ECH:(k + 1) * ECH] for k in range(NCHUNK)]
    row_s3 = [r.reshape(NS, ECH // NS // IDXB, IDXB) for r in row_pc]
    row_w3 = [r.reshape(NW, ECH // NW // IDXB, IDXB) for r in row_pc]
    row_w3f = row_pad.reshape(NW, EPAD // NW // IDXB, IDXB)
    batch_pad = _pad_rows(batch.astype(jnp.int32), NPAD, G)
    bat2_s3 = jnp.concatenate([batch_pad, batch_pad]).reshape(
        NS, 2 * NPAD // NS // IDXB, IDXB)
    nbp = 2 * NCHUNK
    batn_s3 = jnp.concatenate([batch_pad] * nbp).reshape(
        NS, nbp * NPAD // NS // IDXB, IDXB)
    batc_s3 = jnp.concatenate([batch_pad] * NCHUNK).reshape(
        NS, NCHUNK * NPAD // NS // IDXB, IDXB)
    zeros_hbm = jnp.zeros((NPAD, 128), jnp.float32)

    ones_t = jnp.ones((IDXB, 128), jnp.float32)
    ncnt2 = _sc_scatter_add(EPAD, 128, NPAD, 'ones')(
        ones_t, row_w3f, zeros_hbm)
    gcnt128 = _scatter(ncnt2, bat2_s3, zeros_hbm, GPAD, 'single')
    ncnt = ncnt2[:N, 0] + ncnt2[NPAD:NPAD + N, 0]
    gcnt = gcnt128[:G, 0]
    ninv = _pad_rows((1.0 / jnp.maximum(ncnt, 1.0)).reshape(N, 1), NPAD)
    nmask = _pad_rows((ncnt > 0).astype(jnp.float32).reshape(N, 1), NPAD)
    ginv = (1.0 / jnp.maximum(gcnt, 1.0)).reshape(G, 1)
    gmask = (gcnt > 0).astype(jnp.float32).reshape(G, 1)

    x_pad = _pad_rows(x, NPAD)
    ea_pad = _pad_rows(edge_attr, EPAD)
    eac = [ea_pad[k * ECH:(k + 1) * ECH] for k in range(NCHUNK)]
    u = jnp.zeros((G, 64), jnp.float32)

    def seg_graph(e_chunks):
        if e_chunks[0].shape[1] == 128:
            Se = [_scatter(ec, row_w3[k], zeros_hbm, NPAD, 'esplit2')
                  for k, ec in enumerate(e_chunks)]
            Sg = _scatter(jnp.concatenate(Se), batn_s3, zeros_hbm,
                          GPAD, 'single')
        else:
            Se = [_scatter(ec, row_s3[k], zeros_hbm, NPAD, 'fsplit')
                  for k, ec in enumerate(e_chunks)]
            Sg = _scatter(jnp.concatenate(Se), batc_s3, zeros_hbm,
                          GPAD, 'fsplit')
        return Sg[:G]

    def layer(xp, einc, Aw, Abe, li, sx, tx, last):
        dx = xp.shape[1]
        W1 = p['e%d_1_w' % li]; b1 = p['e%d_1_b' % li]
        h = W1.shape[1]
        hp = max(h, 128)
        Ws, Wd = W1[:dx], W1[dx:2 * dx]
        W2 = p['e%d_2_w' % li]; b2 = p['e%d_2_b' % li]
        Ws_f = _pad_cols(sx[:, None] * Ws, hp)
        Wd_f = _pad_cols(sx[:, None] * Wd, hp)
        b1_f = _pad_cols((b1 + tx @ Ws + tx @ Wd)[None, :], hp)[0] + Abe
        W2_p = _pad_rows(W2, hp)

        As = _fused_mm(xp, Ws_f)
        if last:
            Ad = _fused_mm(xp, Wd_f)
            wn = 0
        else:
            V1 = p['n%d_m1a_w' % li]; c1 = p['n%d_m1a_b' % li]
            Vx, Ve = V1[:dx], V1[dx:]
            wn = Ve.shape[1]
            CAd = _fused_mm(xp, jnp.concatenate([sx[:, None] * Vx, Wd_f],
                                                axis=1))
            W2V = W2_p @ Ve
            c1_f = c1 + tx @ Vx + b2 @ Ve

        e_h_c, S_parts, gram_c = [], [], []
        for kc in range(NCHUNK):
            if last:
                OUT = _gab(As, Ad, row_gc[kc], col_gc[kc])
                e_h = _fused_mm(einc[kc], Aw, bias=b1_f, D=OUT, relu=True)
                e_h_c.append(e_h)
                continue
            OUT = _gab(As, CAd, row_gc[kc], col_gc[kc], wn)
            e_h, h1, gk = _edge_mm(einc[kc], Aw, b1_f, OUT, W2V, c1_f,
                                   E - kc * ECH)
            e_h_c.append(e_h)
            gram_c.append(gk)
            if h1.shape[1] == 128:
                S2 = _scatter(h1, row_w3[kc], zeros_hbm, NPAD, 'esplit2')
                S_parts += [S2[:NPAD], S2[NPAD:]]
            else:
                S_parts.append(_scatter(h1, row_s3[kc], zeros_hbm,
                                        NPAD, 'fsplit'))

        Sg_eh = seg_graph(e_h_c)
        g_agg = _fused_mm(Sg_eh, W2_p, bias=b2, arow=ginv,
                          rowmask=gmask, bm=64)
        if last:
            return None, None, g_agg, (W2_p, b2), None

        V2 = p['n%d_m1b_w' % li]; c2 = p['n%d_m1b_b' % li]
        agg = _fused_mm(S_parts[0], V2, bias=c2, Aextra=tuple(S_parts[1:]),
                        arow=ninv, rowmask=nmask)

        M1 = p['n%d_m2a_w' % li]; d1 = p['n%d_m2a_b' % li]
        Mx, Ma = M1[:dx], M1[dx:]
        Mx_f = sx[:, None] * Mx
        d1_f = d1 + tx @ Mx
        nh = _fused_mm(xp, Mx_f, B=agg, Wb=Ma, bias=d1_f, relu=True)
        n_out = _fused_mm(nh, p['n%d_m2b_w' % li], bias=p['n%d_m2b_b' % li])
        gram = gram_c[0]
        for gk in gram_c[1:]:
            gram = gram + gk
        return n_out, e_h_c, g_agg, (W2_p, b2), gram

    def gmlp(u, g_agg, li):
        G1 = p['g%d_1_w' % li]; gb1 = p['g%d_1_b' % li]
        Gu, Gg = G1[:u.shape[1]], G1[u.shape[1]:]
        uh = _fused_mm(u, Gu, B=g_agg, Wb=Gg, bias=gb1, relu=True, bm=64)
        return _fused_mm(uh, p['g%d_2_w' % li], bias=p['g%d_2_b' % li], bm=64)

    def bn_affine(h_raw, mrows, name):
        st = _col_stats(h_raw, mrows)
        mu = st[0] / mrows
        var = st[1] / mrows - mu * mu
        s = p[name + '_g'] * lax.rsqrt(var + 1e-5)
        t = p[name + '_b'] - mu * s
        return s, t

    def bn_from_gram(gs, W2_p, b2, name):
        K = W2_p.shape[0]
        mu_h = gs[K] / E
        T = gs[:K] @ (W2_p / E)
        muy = mu_h @ W2_p
        var = jnp.sum(W2_p * T, axis=0) - muy * muy
        s = p[name + '_g'] * lax.rsqrt(var + 1e-5)
        t = p[name + '_b'] - (muy + b2) * s
        return s, t

    We1 = p['e1_1_w'][2 * 128:]
    Aw1 = _pad_cols(We1, 128)
    x1r, eh1_c, g_agg1, (W2p1, b21), gram1 = layer(
        x_pad, eac, Aw1, jnp.zeros((128,)), 1,
        jnp.ones((128,)), jnp.zeros((128,)), False)
    u = gmlp(u, g_agg1, 1)
    sx1, tx1 = bn_affine(x1r, N, 'bn_n1')
    se1, te1 = bn_from_gram(gram1, W2p1, b21, 'bn_e1')
    We2 = p['e2_1_w'][2 * 256:]
    We2_s = se1[:, None] * We2
    Aw2 = W2p1 @ _pad_cols(We2_s, 128)
    Abe2 = _pad_cols((te1 @ We2 + b21 @ We2_s)[None, :], 128)[0]
    Abe2 = Abe2 + g_agg1[0, 0] * 0.0
    x2r, eh2_c, g_agg2, (W2p2, b22), gram2 = layer(
        x1r, eh1_c, Aw2, Abe2, 2, sx1, tx1, False)
    u = gmlp(u, g_agg2, 2)
    sx2, tx2 = bn_affine(x2r, N, 'bn_n2')
    se2, te2 = bn_from_gram(gram2, W2p2, b22, 'bn_e2')
    We3 = p['e3_1_w'][2 * 512:]
    We3_s = se2[:, None] * We3
    Aw3 = W2p2 @ We3_s
    Abe3 = te2 @ We3 + b22 @ We3_s + g_agg2[0, 0] * 0.0
    _, _, g_agg3, _, _ = layer(x2r, eh2_c, Aw3, Abe3, 3, sx2, tx2, True)
    ea_term = g_agg2 * se2[None, :] + gmask * te2[None, :]
    u = gmlp(u, g_agg3 + ea_term, 3)

    h = _fused_mm(u, p['fc1_w'], bias=p['fc1_b'], relu=True, bm=64)
    return _fused_mm(h, p['fc2_w'], bias=p['fc2_b'], bm=64)
